# Initial kernel scaffold; baseline (speedup 1.0000x reference)
import functools

import jax
import jax.numpy as jnp
from jax import lax
from jax.experimental import pallas as pl
from jax.experimental.pallas import tpu as pltpu
from jax.experimental.pallas import tpu_sc as plsc

N_USERS = 4000
N_ITEMS = 6000
N = N_USERS + N_ITEMS
NP = 10240
D = 128
E = 320000
NC, NS = 2, 16
NW = NC * NS
CH = 80
EPW = E // NW
EPS = E // NS
RPT = NP // NS

_MESH = dict(core_axis_name="c", subcore_axis_name="s", num_cores=NC,
             num_subcores=NS)
_f32 = jnp.float32


_RB = 1000


def _dae_body(x_ref, ucw, ucb, upw, upb, ud1w, ud1b, ud2w, ud2b,
              vcw, vcb, vpw, vpb, vd1w, vd1b, vd2w, vd2b,
              w1, b1, w2, b2,
              int_ref, a_ref, b_ref, c_ref, cor_ref):
    i = pl.program_id(0)
    is_u = i < (N_USERS // _RB)
    x = x_ref[...]

    def pick(u, v):
        return jnp.where(is_u, u[...], v[...])

    cw, cb = pick(ucw, vcw), pick(ucb, vcb)
    pw, pb = pick(upw, vpw), pick(upb, vpb)
    d1w, d1b = pick(ud1w, vd1w), pick(ud1b, vd1b)
    d2w, d2b = pick(ud2w, vd2w), pick(ud2b, vd2b)

    c_ = jnp.dot(x, cw, preferred_element_type=_f32) + cb
    p_ = jnp.dot(x, pw, preferred_element_type=_f32) + pb
    r = jnp.dot(jnp.concatenate([c_, p_], axis=1), d1w,
                preferred_element_type=_f32) + d1b
    r = jnp.dot(jnp.maximum(r, 0.0), d2w, preferred_element_type=_f32) + d2b
    intb = r + x
    int_ref[...] = intb

    d2 = jnp.sum((r - x) ** 2)
    contrib = jnp.where(is_u, jnp.array([[1.0, 0.0]], _f32),
                        jnp.array([[0.0, 1.0]], _f32)) * d2

    @pl.when(i == 0)
    def _():
        cor_ref[...] = jnp.zeros_like(cor_ref)

    cor_ref[...] += contrib

    nrm = jnp.sqrt(jnp.sum(intb * intb, axis=1, keepdims=True))
    y = intb / jnp.maximum(nrm, 1e-12)
    w1v = w1[...]
    a_ref[...] = jnp.dot(y, w1v[:D], preferred_element_type=_f32) + b1[...]
    b_ref[...] = jnp.dot(y, w1v[D:], preferred_element_type=_f32)
    c_ref[...] = (jnp.dot(y, w2[...], preferred_element_type=_f32)
                  + 0.5 * b2[...])


def _dae_precompute(x, uw, vw, w1, b1, w2, b2):
    full = lambda s: pl.BlockSpec(s, lambda i: tuple(0 for _ in s))
    row = lambda k: pl.BlockSpec((_RB, k), lambda i: (i, 0))
    in_specs = [row(D)]
    for w in (uw + vw):
        in_specs.append(full(w.shape))
    in_specs += [full(w1.shape), full(b1.shape), full(w2.shape),
                 full(b2.shape)]
    out_shape = (
        jax.ShapeDtypeStruct((N, D), _f32),
        jax.ShapeDtypeStruct((N, D), _f32),
        jax.ShapeDtypeStruct((N, D), _f32),
        jax.ShapeDtypeStruct((N, 1), _f32),
        jax.ShapeDtypeStruct((1, 2), _f32),
    )
    out_specs = (row(D), row(D), row(D), row(1),
                 pl.BlockSpec((1, 2), lambda i: (0, 0)))
    return pl.pallas_call(
        _dae_body, grid=(N // _RB,), in_specs=in_specs,
        out_specs=out_specs, out_shape=out_shape,
    )(x, *uw, *vw, w1, b1, w2, b2)


def _deg_body(h_hbm, zn_hbm, deg2_hbm, hv, ones_v, acc):
    cid = lax.axis_index("c")
    sid = lax.axis_index("s")
    wid = sid * NC + cid

    for g in range(CH // 16):
        ones_v[pl.ds(g * 16, 16)] = jnp.ones((16,), _f32)
    pltpu.sync_copy(zn_hbm.at[pl.ds(sid * RPT, RPT)],
                    acc.at[pl.ds(sid * RPT, RPT)])
    plsc.subcore_barrier()

    def chunk(i, _):
        base = wid * EPW + i * CH
        pltpu.sync_copy(h_hbm.at[pl.ds(base, CH)], hv)
        pltpu.sync_copy(ones_v, acc.at[hv], add=True)
        return 0

    lax.fori_loop(0, EPW // CH, chunk, 0)
    plsc.subcore_barrier()
    pltpu.sync_copy(acc.at[pl.ds(sid * RPT, RPT)],
                    deg2_hbm.at[cid, pl.ds(sid * RPT, RPT)])


def _deg(h, zn):
    return pl.kernel(
        _deg_body,
        out_type=jax.ShapeDtypeStruct((NC, NP), _f32),
        mesh=plsc.VectorSubcoreMesh(**_MESH),
        scratch_types=[
            pltpu.VMEM((CH,), jnp.int32),
            pltpu.VMEM((CH,), _f32),
            pltpu.VMEM_SHARED((NP,), _f32),
        ],
    )(h, zn)


def _alpha_body(a_hbm, b_hbm, c_hbm, w_hbm, h_hbm, t_hbm, zn_hbm,
                alpha_hbm, dsum2_hbm,
                hv, tv, arows, brows, av, cv, wv, acc, sema, semb):
    cid = lax.axis_index("c")
    sid = lax.axis_index("s")
    wid = sid * NC + cid

    pltpu.sync_copy(c_hbm, cv)
    pltpu.sync_copy(w_hbm, wv)
    pltpu.sync_copy(zn_hbm.at[pl.ds(sid * RPT, RPT)],
                    acc.at[pl.ds(sid * RPT, RPT)])
    plsc.subcore_barrier()

    def chunk(i, _):
        base = wid * EPW + i * CH
        pltpu.sync_copy(h_hbm.at[pl.ds(base, CH)], hv)
        pltpu.sync_copy(t_hbm.at[pl.ds(base, CH)], tv)
        cpa = pltpu.async_copy(a_hbm.at[hv], arows, sema)
        cpb = pltpu.async_copy(b_hbm.at[tv], brows, semb)
        cpa.wait()
        cpb.wait()
        for g in range(CH // 16):
            rowi = (jnp.full((16,), g * 16, jnp.int32)
                    + lax.iota(jnp.int32, 16))

            def col(j, acc16):
                colv = jnp.full((16,), j, jnp.int32)
                va = plsc.load_gather(arows, [rowi, colv])
                vb = plsc.load_gather(brows, [rowi, colv])
                z = va + vb
                lr = jnp.maximum(z, 0.0) + 0.2 * jnp.minimum(z, 0.0)
                return acc16 + lr * wv[j]

            acc16 = lax.fori_loop(0, D, col, jnp.zeros((16,), _f32))
            hg = hv[pl.ds(g * 16, 16)]
            tg = tv[pl.ds(g * 16, 16)]
            s = (acc16 + plsc.load_gather(cv, [hg])
                 + plsc.load_gather(cv, [tg]))
            av[pl.ds(g * 16, 16)] = 1.0 / (1.0 + jnp.exp(-s))
        pltpu.sync_copy(av, alpha_hbm.at[pl.ds(base, CH)])
        pltpu.sync_copy(av, acc.at[hv], add=True)
        return 0

    lax.fori_loop(0, EPW // CH, chunk, 0)
    plsc.subcore_barrier()
    pltpu.sync_copy(acc.at[pl.ds(sid * RPT, RPT)],
                    dsum2_hbm.at[cid, pl.ds(sid * RPT, RPT)])


def _alpha(a, b, c, w, h, t, zn):
    return pl.kernel(
        _alpha_body,
        out_type=(jax.ShapeDtypeStruct((E,), _f32),
                  jax.ShapeDtypeStruct((NC, NP), _f32)),
        mesh=plsc.VectorSubcoreMesh(**_MESH),
        scratch_types=[
            pltpu.VMEM((CH,), jnp.int32),
            pltpu.VMEM((CH,), jnp.int32),
            pltpu.VMEM((CH, D), _f32),
            pltpu.VMEM((CH, D), _f32),
            pltpu.VMEM((CH,), _f32),
            pltpu.VMEM((NP,), _f32),
            pltpu.VMEM((D,), _f32),
            pltpu.VMEM_SHARED((NP,), _f32),
            pltpu.SemaphoreType.DMA,
            pltpu.SemaphoreType.DMA,
        ],
    )(a, b, c, w, h, t, zn)


def _spmm_body(x_hbm, h_hbm, t_hbm, alpha_hbm, dinv_hbm, dsinv_hbm, zr_hbm,
               gnn_hbm, iaa_hbm,
               hv, tv, av, valv, rows, dinv_v, dsinv_v, acc, sem):
    cid = lax.axis_index("c")
    sid = lax.axis_index("s")

    pltpu.sync_copy(dinv_hbm, dinv_v)
    pltpu.sync_copy(dsinv_hbm, dsinv_v)
    pltpu.sync_copy(zr_hbm, acc.at[pl.ds(sid * RPT, RPT)])
    plsc.subcore_barrier()

    def chunk(i, _):
        base = sid * EPS + i * CH
        pltpu.sync_copy(h_hbm.at[pl.ds(base, CH)], hv)
        pltpu.sync_copy(t_hbm.at[pl.ds(base, CH)], tv)

        @pl.when(cid == 1)
        def _():
            pltpu.sync_copy(alpha_hbm.at[pl.ds(base, CH)], av)

        pltpu.async_copy(x_hbm.at[tv], rows, sem).wait()

        for g in range(CH // 16):
            hg = hv[pl.ds(g * 16, 16)]
            tg = tv[pl.ds(g * 16, 16)]

            @pl.when(cid == 0)
            def _():
                valv[pl.ds(g * 16, 16)] = (plsc.load_gather(dinv_v, [hg])
                                           * plsc.load_gather(dinv_v, [tg]))

            @pl.when(cid == 1)
            def _():
                valv[pl.ds(g * 16, 16)] = (plsc.load_gather(dsinv_v, [hg])
                                           * av[pl.ds(g * 16, 16)])

        def scale(e, _):
            sc = valv[e]
            for j in range(D // 16):
                rows[e, pl.ds(j * 16, 16)] = rows[e, pl.ds(j * 16, 16)] * sc
            return 0

        lax.fori_loop(0, CH, scale, 0)
        pltpu.sync_copy(rows, acc.at[hv], add=True)
        return 0

    lax.fori_loop(0, EPS // CH, chunk, 0)
    plsc.subcore_barrier()

    @pl.when(cid == 0)
    def _():
        pltpu.sync_copy(acc.at[pl.ds(sid * RPT, RPT)],
                        gnn_hbm.at[pl.ds(sid * RPT, RPT)])

    @pl.when(cid == 1)
    def _():
        pltpu.sync_copy(acc.at[pl.ds(sid * RPT, RPT)],
                        iaa_hbm.at[pl.ds(sid * RPT, RPT)])


def _spmm(x, h, t, alpha, dinv, dsinv, zr):
    return pl.kernel(
        _spmm_body,
        out_type=(jax.ShapeDtypeStruct((NP, D), _f32),
                  jax.ShapeDtypeStruct((NP, D), _f32)),
        mesh=plsc.VectorSubcoreMesh(**_MESH),
        scratch_types=[
            pltpu.VMEM((CH,), jnp.int32),
            pltpu.VMEM((CH,), jnp.int32),
            pltpu.VMEM((CH,), _f32),
            pltpu.VMEM((CH,), _f32),
            pltpu.VMEM((CH, D), _f32),
            pltpu.VMEM((NP,), _f32),
            pltpu.VMEM((NP,), _f32),
            pltpu.VMEM_SHARED((NP, D), _f32),
            pltpu.SemaphoreType.DMA,
        ],
    )(x, h, t, alpha, dinv, dsinv, zr)


def kernel(all_h_list, all_t_list, user_emb, item_emb, W1, b1, W2, b2,
           uCW, uCb, uPW, uPb, uD1W, uD1b, uD2W, uD2b,
           vCW, vCb, vPW, vPb, vD1W, vD1b, vD2W, vD2b):
    h = all_h_list
    t = all_t_list
    zn = jnp.zeros((NP,), _f32)
    zr = jnp.zeros((RPT, D), _f32)

    deg2 = _deg(h, zn)
    deg = deg2[0] + deg2[1]
    dinv = jnp.where(deg > 0, lax.rsqrt(jnp.maximum(deg, 1e-30)), 0.0)

    uw = (uCW, uCb.reshape(1, -1), uPW, uPb.reshape(1, -1),
          uD1W, uD1b.reshape(1, -1), uD2W, uD2b.reshape(1, -1))
    vw = (vCW, vCb.reshape(1, -1), vPW, vPb.reshape(1, -1),
          vD1W, vD1b.reshape(1, -1), vD2W, vD2b.reshape(1, -1))

    x = jnp.concatenate([user_emb, item_emb], axis=0)
    x0 = x
    gnn_l, int_l, iaa_l = [], [], []
    cor = jnp.float32(0.0)

    for _ in range(2):
        int_layer, a_mat, b_mat, c_mat, cor2 = _dae_precompute(
            x, uw, vw, W1, b1.reshape(1, -1), W2, b2.reshape(1, -1))
        cor = cor + cor2[0, 0] / (N_USERS * D) + cor2[0, 1] / (N_ITEMS * D)
        c_pad = jnp.pad(c_mat[:, 0], (0, NP - N))
        alpha, dsum2 = _alpha(a_mat, b_mat, c_pad, W2[:, 0], h, t, zn)
        dsum = dsum2[0] + dsum2[1]
        dsinv = jnp.where(dsum != 0, 1.0 / jnp.where(dsum != 0, dsum, 1.0),
                          0.0)
        gnn_p, iaa_p = _spmm(x, h, t, alpha, dinv, dsinv, zr)
        gnn, iaa = gnn_p[:N], iaa_p[:N]
        gnn_l.append(gnn)
        int_l.append(int_layer)
        iaa_l.append(iaa)
        x = gnn + iaa + x

    final = x0 + gnn_l[0] + iaa_l[0] + x
    return (jnp.stack(gnn_l), jnp.stack(int_l), jnp.stack(iaa_l), final, cor)


# speedup vs baseline: 5.8028x; 5.8028x over previous
import functools

import jax
import jax.numpy as jnp
from jax import lax
from jax.experimental import pallas as pl
from jax.experimental.pallas import tpu as pltpu
from jax.experimental.pallas import tpu_sc as plsc

N_USERS = 4000
N_ITEMS = 6000
N = N_USERS + N_ITEMS
NP = 10240
D = 128
E = 320000
NC, NS = 2, 16
NW = NC * NS
CH = 80
EPW = E // NW
EPS = E // NS
RPT = NP // NS

_MESH = dict(core_axis_name="c", subcore_axis_name="s", num_cores=NC,
             num_subcores=NS)
_f32 = jnp.float32


_RB = 1000


def _dae_body(x_ref, ucw, ucb, upw, upb, ud1w, ud1b, ud2w, ud2b,
              vcw, vcb, vpw, vpb, vd1w, vd1b, vd2w, vd2b,
              w1, b1, w2, b2,
              int_ref, a_ref, b_ref, c_ref, cor_ref):
    i = pl.program_id(0)
    is_u = i < (N_USERS // _RB)
    x = x_ref[...]

    def pick(u, v):
        return jnp.where(is_u, u[...], v[...])

    cw, cb = pick(ucw, vcw), pick(ucb, vcb)
    pw, pb = pick(upw, vpw), pick(upb, vpb)
    d1w, d1b = pick(ud1w, vd1w), pick(ud1b, vd1b)
    d2w, d2b = pick(ud2w, vd2w), pick(ud2b, vd2b)

    c_ = jnp.dot(x, cw, preferred_element_type=_f32) + cb
    p_ = jnp.dot(x, pw, preferred_element_type=_f32) + pb
    r = jnp.dot(jnp.concatenate([c_, p_], axis=1), d1w,
                preferred_element_type=_f32) + d1b
    r = jnp.dot(jnp.maximum(r, 0.0), d2w, preferred_element_type=_f32) + d2b
    intb = r + x
    int_ref[...] = intb

    d2 = jnp.sum((r - x) ** 2)
    lane = lax.broadcasted_iota(jnp.int32, (1, 2), 1)
    contrib = jnp.where(lane == jnp.where(is_u, 0, 1), d2, 0.0)

    @pl.when(i == 0)
    def _():
        cor_ref[...] = jnp.zeros_like(cor_ref)

    cor_ref[...] += contrib

    nrm = jnp.sqrt(jnp.sum(intb * intb, axis=1, keepdims=True))
    y = intb / jnp.maximum(nrm, 1e-12)
    w1v = w1[...]
    a_ref[...] = jnp.dot(y, w1v[:D], preferred_element_type=_f32) + b1[...]
    b_ref[...] = jnp.dot(y, w1v[D:], preferred_element_type=_f32)
    c_ref[...] = (jnp.dot(y, w2[...], preferred_element_type=_f32)
                  + 0.5 * b2[...])


def _dae_precompute(x, uw, vw, w1, b1, w2, b2):
    full = lambda s: pl.BlockSpec(s, lambda i: tuple(0 for _ in s))
    row = lambda k: pl.BlockSpec((_RB, k), lambda i: (i, 0))
    in_specs = [row(D)]
    for w in (uw + vw):
        in_specs.append(full(w.shape))
    in_specs += [full(w1.shape), full(b1.shape), full(w2.shape),
                 full(b2.shape)]
    out_shape = (
        jax.ShapeDtypeStruct((N, D), _f32),
        jax.ShapeDtypeStruct((N, D), _f32),
        jax.ShapeDtypeStruct((N, D), _f32),
        jax.ShapeDtypeStruct((N, 1), _f32),
        jax.ShapeDtypeStruct((1, 2), _f32),
    )
    out_specs = (row(D), row(D), row(D), row(1),
                 pl.BlockSpec((1, 2), lambda i: (0, 0)))
    return pl.pallas_call(
        _dae_body, grid=(N // _RB,), in_specs=in_specs,
        out_specs=out_specs, out_shape=out_shape,
    )(x, *uw, *vw, w1, b1, w2, b2)


def _deg_body(h_hbm, zn_hbm, deg2_hbm, hv, ones_v, acc):
    cid = lax.axis_index("c")
    sid = lax.axis_index("s")
    wid = sid * NC + cid

    for g in range(CH // 16):
        ones_v[pl.ds(g * 16, 16)] = jnp.ones((16,), _f32)
    pltpu.sync_copy(zn_hbm.at[pl.ds(sid * RPT, RPT)],
                    acc.at[pl.ds(sid * RPT, RPT)])
    plsc.subcore_barrier()

    def chunk(i, _):
        base = wid * EPW + i * CH
        pltpu.sync_copy(h_hbm.at[pl.ds(base, CH)], hv)
        pltpu.sync_copy(ones_v, acc.at[hv], add=True)
        return 0

    lax.fori_loop(0, EPW // CH, chunk, 0)
    plsc.subcore_barrier()
    pltpu.sync_copy(acc.at[pl.ds(sid * RPT, RPT)],
                    deg2_hbm.at[cid, pl.ds(sid * RPT, RPT)])


def _deg(h, zn):
    return pl.kernel(
        _deg_body,
        out_type=jax.ShapeDtypeStruct((NC, NP), _f32),
        mesh=plsc.VectorSubcoreMesh(**_MESH),
        scratch_types=[
            pltpu.VMEM((CH,), jnp.int32),
            pltpu.VMEM((CH,), _f32),
            pltpu.VMEM_SHARED((NP,), _f32),
        ],
    )(h, zn)


def _alpha_body(a_hbm, b_hbm, c_hbm, w_hbm, h_hbm, t_hbm, zn_hbm,
                alpha_hbm, dsum2_hbm,
                hv, tv, arows, brows, av, chv, ctv, wv,
                acc, sema, semb, semc, semd):
    cid = lax.axis_index("c")
    sid = lax.axis_index("s")
    wid = sid * NC + cid

    pltpu.sync_copy(w_hbm, wv)
    pltpu.sync_copy(zn_hbm.at[pl.ds(sid * RPT, RPT)],
                    acc.at[pl.ds(sid * RPT, RPT)])
    plsc.subcore_barrier()

    lane = lax.iota(jnp.int32, 16)

    def chunk(i, _):
        base = wid * EPW + i * CH
        pltpu.sync_copy(h_hbm.at[pl.ds(base, CH)], hv)
        pltpu.sync_copy(t_hbm.at[pl.ds(base, CH)], tv)
        cpa = pltpu.async_copy(a_hbm.at[hv], arows, sema)
        cpb = pltpu.async_copy(b_hbm.at[tv], brows, semb)
        cpc = pltpu.async_copy(c_hbm.at[hv], chv, semc)
        cpd = pltpu.async_copy(c_hbm.at[tv], ctv, semd)
        cpa.wait()
        cpb.wait()
        cpc.wait()
        cpd.wait()
        ws = [wv[pl.ds(j * 16, 16)] for j in range(D // 16)]

        def group(g, _):
            s16 = jnp.zeros((16,), _f32)
            for e16 in range(16):
                e = g * 16 + e16
                vacc = jnp.zeros((16,), _f32)
                for j in range(D // 16):
                    z = (arows[e, pl.ds(j * 16, 16)]
                         + brows[e, pl.ds(j * 16, 16)])
                    lr = jnp.maximum(z, 0.0) + 0.2 * jnp.minimum(z, 0.0)
                    vacc = vacc + lr * ws[j]
                for sh in (8, 4, 2, 1):
                    perm = jnp.bitwise_and(lane + sh, 15)
                    vacc = vacc + vacc.at[perm].get(mode="promise_in_bounds")
                s16 = jnp.where(lane == e16, vacc, s16)
            sl = pl.ds(g * 16, 16)
            s16 = s16 + chv[sl] + ctv[sl]
            av[sl] = 1.0 / (1.0 + jnp.exp(-s16))
            return 0

        lax.fori_loop(0, CH // 16, group, 0)
        pltpu.sync_copy(av, alpha_hbm.at[pl.ds(base, CH)])
        pltpu.sync_copy(av, acc.at[hv], add=True)
        return 0

    lax.fori_loop(0, EPW // CH, chunk, 0)
    plsc.subcore_barrier()
    pltpu.sync_copy(acc.at[pl.ds(sid * RPT, RPT)],
                    dsum2_hbm.at[cid, pl.ds(sid * RPT, RPT)])


def _alpha(a, b, c, w, h, t, zn):
    return pl.kernel(
        _alpha_body,
        out_type=(jax.ShapeDtypeStruct((E,), _f32),
                  jax.ShapeDtypeStruct((NC, NP), _f32)),
        mesh=plsc.VectorSubcoreMesh(**_MESH),
        scratch_types=[
            pltpu.VMEM((CH,), jnp.int32),
            pltpu.VMEM((CH,), jnp.int32),
            pltpu.VMEM((CH, D), _f32),
            pltpu.VMEM((CH, D), _f32),
            pltpu.VMEM((CH,), _f32),
            pltpu.VMEM((CH,), _f32),
            pltpu.VMEM((CH,), _f32),
            pltpu.VMEM((D,), _f32),
            pltpu.VMEM_SHARED((NP,), _f32),
            pltpu.SemaphoreType.DMA,
            pltpu.SemaphoreType.DMA,
            pltpu.SemaphoreType.DMA,
            pltpu.SemaphoreType.DMA,
        ],
    )(a, b, c, w, h, t, zn)


def _spmm_body(x_hbm, h_hbm, t_hbm, alpha_hbm, dinv_hbm, dsinv_hbm, zr_hbm,
               out_hbm,
               hv, tv, av, g0h, g0t, g1h, valv, rows, acc,
               sem, semg, semh, semi):
    cid = lax.axis_index("c")
    sid = lax.axis_index("s")

    pltpu.sync_copy(zr_hbm, acc.at[pl.ds(sid * RPT, RPT)])
    plsc.subcore_barrier()
    is0 = cid == 0

    def chunk(i, _):
        base = sid * EPS + i * CH
        pltpu.sync_copy(h_hbm.at[pl.ds(base, CH)], hv)
        pltpu.sync_copy(t_hbm.at[pl.ds(base, CH)], tv)
        pltpu.sync_copy(alpha_hbm.at[pl.ds(base, CH)], av)
        cpr = pltpu.async_copy(x_hbm.at[tv], rows, sem)
        cpg = pltpu.async_copy(dinv_hbm.at[hv], g0h, semg)
        cpt = pltpu.async_copy(dinv_hbm.at[tv], g0t, semh)
        cps = pltpu.async_copy(dsinv_hbm.at[hv], g1h, semi)
        cpg.wait()
        cpt.wait()
        cps.wait()
        cpr.wait()

        for g in range(CH // 16):
            sl = pl.ds(g * 16, 16)
            valv[sl] = jnp.where(is0, g0h[sl] * g0t[sl], g1h[sl] * av[sl])

        def scale(e, _):
            sc = valv[pl.ds(e, 16)][0]
            for j in range(D // 16):
                rows[e, pl.ds(j * 16, 16)] = rows[e, pl.ds(j * 16, 16)] * sc
            return 0

        lax.fori_loop(0, CH, scale, 0)
        pltpu.sync_copy(rows, acc.at[hv], add=True)
        return 0

    lax.fori_loop(0, EPS // CH, chunk, 0)
    plsc.subcore_barrier()
    pltpu.sync_copy(acc.at[pl.ds(sid * RPT, RPT)],
                    out_hbm.at[cid, pl.ds(sid * RPT, RPT)])


def _spmm(x, h, t, alpha, dinv, dsinv, zr):
    return pl.kernel(
        _spmm_body,
        out_type=jax.ShapeDtypeStruct((NC, NP, D), _f32),
        mesh=plsc.VectorSubcoreMesh(**_MESH),
        scratch_types=[
            pltpu.VMEM((CH,), jnp.int32),
            pltpu.VMEM((CH,), jnp.int32),
            pltpu.VMEM((CH,), _f32),
            pltpu.VMEM((CH,), _f32),
            pltpu.VMEM((CH,), _f32),
            pltpu.VMEM((CH,), _f32),
            pltpu.VMEM((CH + 16,), _f32),
            pltpu.VMEM((CH, D), _f32),
            pltpu.VMEM_SHARED((NP, D), _f32),
            pltpu.SemaphoreType.DMA,
            pltpu.SemaphoreType.DMA,
            pltpu.SemaphoreType.DMA,
            pltpu.SemaphoreType.DMA,
        ],
    )(x, h, t, alpha, dinv, dsinv, zr)


def kernel(all_h_list, all_t_list, user_emb, item_emb, W1, b1, W2, b2,
           uCW, uCb, uPW, uPb, uD1W, uD1b, uD2W, uD2b,
           vCW, vCb, vPW, vPb, vD1W, vD1b, vD2W, vD2b):
    h = all_h_list
    t = all_t_list
    zn = jnp.zeros((NP,), _f32)
    zr = jnp.zeros((RPT, D), _f32)

    deg2 = _deg(h, zn)
    deg = deg2[0] + deg2[1]
    dinv = jnp.where(deg > 0, lax.rsqrt(jnp.maximum(deg, 1e-30)), 0.0)

    uw = (uCW, uCb.reshape(1, -1), uPW, uPb.reshape(1, -1),
          uD1W, uD1b.reshape(1, -1), uD2W, uD2b.reshape(1, -1))
    vw = (vCW, vCb.reshape(1, -1), vPW, vPb.reshape(1, -1),
          vD1W, vD1b.reshape(1, -1), vD2W, vD2b.reshape(1, -1))

    x = jnp.concatenate([user_emb, item_emb], axis=0)
    final = x
    gnn_l, int_l, iaa_l = [], [], []
    cor = jnp.float32(0.0)

    for _ in range(2):
        int_layer, a_mat, b_mat, c_mat, cor2 = _dae_precompute(
            x, uw, vw, W1, b1.reshape(1, -1), W2, b2.reshape(1, -1))
        cor = cor + cor2[0, 0] / (N_USERS * D) + cor2[0, 1] / (N_ITEMS * D)
        c_pad = jnp.pad(c_mat[:, 0], (0, NP - N))
        alpha, dsum2 = _alpha(a_mat, b_mat, c_pad, W2[:, 0], h, t, zn)
        dsum = dsum2[0] + dsum2[1]
        dsinv = jnp.where(dsum != 0, 1.0 / jnp.where(dsum != 0, dsum, 1.0),
                          0.0)
        out2 = _spmm(x, h, t, alpha, dinv, dsinv, zr)
        gnn, iaa = out2[0, :N], out2[1, :N]
        gnn_l.append(gnn)
        int_l.append(int_layer)
        iaa_l.append(iaa)
        x = gnn + iaa + x
        final = final + x

    return (jnp.stack(gnn_l), jnp.stack(int_l), jnp.stack(iaa_l), final, cor)


# speedup vs baseline: 7.2732x; 1.2534x over previous
import functools

import jax
import jax.numpy as jnp
from jax import lax
from jax.experimental import pallas as pl
from jax.experimental.pallas import tpu as pltpu
from jax.experimental.pallas import tpu_sc as plsc

N_USERS = 4000
N_ITEMS = 6000
N = N_USERS + N_ITEMS
NP = 10240
D = 128
E = 320000
NC, NS = 2, 16
NW = NC * NS
CH = 80
EPW = E // NW
EPS = E // NS
RPT = NP // NS

_MESH = dict(core_axis_name="c", subcore_axis_name="s", num_cores=NC,
             num_subcores=NS)
_f32 = jnp.float32


_RB = 1000


def _dae_body(x_ref, ucw, ucb, upw, upb, ud1w, ud1b, ud2w, ud2b,
              vcw, vcb, vpw, vpb, vd1w, vd1b, vd2w, vd2b,
              w1, b1, w2, b2,
              int_ref, a_ref, b_ref, c_ref, cor_ref):
    i = pl.program_id(0)
    is_u = i < (N_USERS // _RB)
    x = x_ref[...]

    def pick(u, v):
        return jnp.where(is_u, u[...], v[...])

    cw, cb = pick(ucw, vcw), pick(ucb, vcb)
    pw, pb = pick(upw, vpw), pick(upb, vpb)
    d1w, d1b = pick(ud1w, vd1w), pick(ud1b, vd1b)
    d2w, d2b = pick(ud2w, vd2w), pick(ud2b, vd2b)

    c_ = jnp.dot(x, cw, preferred_element_type=_f32) + cb
    p_ = jnp.dot(x, pw, preferred_element_type=_f32) + pb
    r = jnp.dot(jnp.concatenate([c_, p_], axis=1), d1w,
                preferred_element_type=_f32) + d1b
    r = jnp.dot(jnp.maximum(r, 0.0), d2w, preferred_element_type=_f32) + d2b
    intb = r + x
    int_ref[...] = intb

    d2 = jnp.sum((r - x) ** 2)
    lane = lax.broadcasted_iota(jnp.int32, (1, 2), 1)
    contrib = jnp.where(lane == jnp.where(is_u, 0, 1), d2, 0.0)

    @pl.when(i == 0)
    def _():
        cor_ref[...] = jnp.zeros_like(cor_ref)

    cor_ref[...] += contrib

    nrm = jnp.sqrt(jnp.sum(intb * intb, axis=1, keepdims=True))
    y = intb / jnp.maximum(nrm, 1e-12)
    w1v = w1[...]
    a_ref[...] = jnp.dot(y, w1v[:D], preferred_element_type=_f32) + b1[...]
    b_ref[...] = jnp.dot(y, w1v[D:], preferred_element_type=_f32)
    c_ref[...] = (jnp.dot(y, w2[...], preferred_element_type=_f32)
                  + 0.5 * b2[...])


def _dae_precompute(x, uw, vw, w1, b1, w2, b2):
    full = lambda s: pl.BlockSpec(s, lambda i: tuple(0 for _ in s))
    row = lambda k: pl.BlockSpec((_RB, k), lambda i: (i, 0))
    in_specs = [row(D)]
    for w in (uw + vw):
        in_specs.append(full(w.shape))
    in_specs += [full(w1.shape), full(b1.shape), full(w2.shape),
                 full(b2.shape)]
    out_shape = (
        jax.ShapeDtypeStruct((N, D), _f32),
        jax.ShapeDtypeStruct((N, D), _f32),
        jax.ShapeDtypeStruct((N, D), _f32),
        jax.ShapeDtypeStruct((N, 1), _f32),
        jax.ShapeDtypeStruct((1, 2), _f32),
    )
    out_specs = (row(D), row(D), row(D), row(1),
                 pl.BlockSpec((1, 2), lambda i: (0, 0)))
    return pl.pallas_call(
        _dae_body, grid=(N // _RB,), in_specs=in_specs,
        out_specs=out_specs, out_shape=out_shape,
    )(x, *uw, *vw, w1, b1, w2, b2)


def _deg_body(h_hbm, zn_hbm, deg2_hbm, hv, ones_v, acc):
    cid = lax.axis_index("c")
    sid = lax.axis_index("s")
    wid = sid * NC + cid

    for g in range(CH // 16):
        ones_v[pl.ds(g * 16, 16)] = jnp.ones((16,), _f32)
    pltpu.sync_copy(zn_hbm.at[pl.ds(sid * RPT, RPT)],
                    acc.at[pl.ds(sid * RPT, RPT)])
    plsc.subcore_barrier()

    def chunk(i, _):
        base = wid * EPW + i * CH
        pltpu.sync_copy(h_hbm.at[pl.ds(base, CH)], hv)
        pltpu.sync_copy(ones_v, acc.at[hv], add=True)
        return 0

    lax.fori_loop(0, EPW // CH, chunk, 0)
    plsc.subcore_barrier()
    pltpu.sync_copy(acc.at[pl.ds(sid * RPT, RPT)],
                    deg2_hbm.at[cid, pl.ds(sid * RPT, RPT)])


def _deg(h, zn):
    return pl.kernel(
        _deg_body,
        out_type=jax.ShapeDtypeStruct((NC, NP), _f32),
        mesh=plsc.VectorSubcoreMesh(**_MESH),
        scratch_types=[
            pltpu.VMEM((CH,), jnp.int32),
            pltpu.VMEM((CH,), _f32),
            pltpu.VMEM_SHARED((NP,), _f32),
        ],
    )(h, zn)


def _alpha_body(a_hbm, b_hbm, c_hbm, w_hbm, h_hbm, t_hbm, zn_hbm,
                alpha_hbm, dsum2_hbm,
                hv, tv, arows, brows, av, chv, ctv, wv,
                acc, sema, semb, semc, semd):
    cid = lax.axis_index("c")
    sid = lax.axis_index("s")
    wid = sid * NC + cid

    pltpu.sync_copy(w_hbm, wv)
    pltpu.sync_copy(zn_hbm.at[pl.ds(sid * RPT, RPT)],
                    acc.at[pl.ds(sid * RPT, RPT)])
    plsc.subcore_barrier()

    lane = lax.iota(jnp.int32, 16)

    def chunk(i, _):
        base = wid * EPW + i * CH
        pltpu.sync_copy(h_hbm.at[pl.ds(base, CH)], hv)
        pltpu.sync_copy(t_hbm.at[pl.ds(base, CH)], tv)
        cpa = pltpu.async_copy(a_hbm.at[hv], arows, sema)
        cpb = pltpu.async_copy(b_hbm.at[tv], brows, semb)
        cpc = pltpu.async_copy(c_hbm.at[hv], chv, semc)
        cpd = pltpu.async_copy(c_hbm.at[tv], ctv, semd)
        cpa.wait()
        cpb.wait()
        cpc.wait()
        cpd.wait()
        ws = [wv[pl.ds(j * 16, 16)] for j in range(D // 16)]

        def group(g, _):
            s16 = jnp.zeros((16,), _f32)
            for e16 in range(16):
                e = g * 16 + e16
                vacc = jnp.zeros((16,), _f32)
                for j in range(D // 16):
                    z = (arows[e, pl.ds(j * 16, 16)]
                         + brows[e, pl.ds(j * 16, 16)])
                    lr = jnp.maximum(z, 0.0) + 0.2 * jnp.minimum(z, 0.0)
                    vacc = vacc + lr * ws[j]
                for sh in (8, 4, 2, 1):
                    perm = jnp.bitwise_and(lane + sh, 15)
                    vacc = vacc + vacc.at[perm].get(mode="promise_in_bounds")
                s16 = jnp.where(lane == e16, vacc, s16)
            sl = pl.ds(g * 16, 16)
            s16 = s16 + chv[sl] + ctv[sl]
            av[sl] = 1.0 / (1.0 + jnp.exp(-s16))
            return 0

        lax.fori_loop(0, CH // 16, group, 0)
        pltpu.sync_copy(av, alpha_hbm.at[pl.ds(base, CH)])
        pltpu.sync_copy(av, acc.at[hv], add=True)
        return 0

    lax.fori_loop(0, EPW // CH, chunk, 0)
    plsc.subcore_barrier()
    pltpu.sync_copy(acc.at[pl.ds(sid * RPT, RPT)],
                    dsum2_hbm.at[cid, pl.ds(sid * RPT, RPT)])


def _alpha(a, b, c, w, h, t, zn):
    return pl.kernel(
        _alpha_body,
        out_type=(jax.ShapeDtypeStruct((E,), _f32),
                  jax.ShapeDtypeStruct((NC, NP), _f32)),
        mesh=plsc.VectorSubcoreMesh(**_MESH),
        scratch_types=[
            pltpu.VMEM((CH,), jnp.int32),
            pltpu.VMEM((CH,), jnp.int32),
            pltpu.VMEM((CH, D), _f32),
            pltpu.VMEM((CH, D), _f32),
            pltpu.VMEM((CH,), _f32),
            pltpu.VMEM((CH,), _f32),
            pltpu.VMEM((CH,), _f32),
            pltpu.VMEM((D,), _f32),
            pltpu.VMEM_SHARED((NP,), _f32),
            pltpu.SemaphoreType.DMA,
            pltpu.SemaphoreType.DMA,
            pltpu.SemaphoreType.DMA,
            pltpu.SemaphoreType.DMA,
        ],
    )(a, b, c, w, h, t, zn)


def _spmm_body(x_hbm, h_hbm, t_hbm, alpha_hbm, dinv_hbm, zr_hbm,
               out_hbm,
               hv0, tv0, av0, gt0, rows0, valv0,
               hv1, tv1, av1, gt1, rows1, valv1,
               acc, semr0, semg0, semr1, semg1):
    cid = lax.axis_index("c")
    sid = lax.axis_index("s")

    pltpu.sync_copy(zr_hbm, acc.at[pl.ds(sid * RPT, RPT)])
    plsc.subcore_barrier()
    is0 = cid == 0
    bufs = ((hv0, tv0, av0, gt0, rows0, valv0, semr0, semg0),
            (hv1, tv1, av1, gt1, rows1, valv1, semr1, semg1))
    nch = EPS // CH

    def load_fire(ci, b):
        hv, tv, av, gt, rows, valv, semr, semg = b
        base = sid * EPS + ci * CH
        pltpu.sync_copy(h_hbm.at[pl.ds(base, CH)], hv)
        pltpu.sync_copy(t_hbm.at[pl.ds(base, CH)], tv)
        pltpu.sync_copy(alpha_hbm.at[pl.ds(base, CH)], av)
        pltpu.async_copy(x_hbm.at[tv], rows, semr)
        pltpu.async_copy(dinv_hbm.at[tv], gt, semg)

    def process(b):
        hv, tv, av, gt, rows, valv, semr, semg = b
        pltpu.make_async_copy(x_hbm.at[tv], rows, semr).wait()
        pltpu.make_async_copy(dinv_hbm.at[tv], gt, semg).wait()
        for g in range(CH // 16):
            sl = pl.ds(g * 16, 16)
            valv[sl] = jnp.where(is0, gt[sl], av[sl])

        def scale(e, _):
            sc = valv[pl.ds(e, 16)][0]
            for j in range(D // 16):
                rows[e, pl.ds(j * 16, 16)] = rows[e, pl.ds(j * 16, 16)] * sc
            return 0

        lax.fori_loop(0, CH, scale, 0)
        pltpu.sync_copy(rows, acc.at[hv], add=True)

    load_fire(0, bufs[0])

    def pair(k, _):
        load_fire(2 * k + 1, bufs[1])
        process(bufs[0])

        @pl.when(k < nch // 2 - 1)
        def _():
            load_fire(2 * k + 2, bufs[0])

        process(bufs[1])
        return 0

    lax.fori_loop(0, nch // 2, pair, 0)
    plsc.subcore_barrier()
    pltpu.sync_copy(acc.at[pl.ds(sid * RPT, RPT)],
                    out_hbm.at[cid, pl.ds(sid * RPT, RPT)])


def _spmm(x, h, t, alpha, dinv, zr):
    buf = [
        pltpu.VMEM((CH,), jnp.int32),
        pltpu.VMEM((CH,), jnp.int32),
        pltpu.VMEM((CH,), _f32),
        pltpu.VMEM((CH,), _f32),
        pltpu.VMEM((CH, D), _f32),
        pltpu.VMEM((CH + 16,), _f32),
    ]
    return pl.kernel(
        _spmm_body,
        out_type=jax.ShapeDtypeStruct((NC, NP, D), _f32),
        mesh=plsc.VectorSubcoreMesh(**_MESH),
        scratch_types=buf + buf + [
            pltpu.VMEM_SHARED((NP, D), _f32),
            pltpu.SemaphoreType.DMA,
            pltpu.SemaphoreType.DMA,
            pltpu.SemaphoreType.DMA,
            pltpu.SemaphoreType.DMA,
        ],
    )(x, h, t, alpha, dinv, zr)


def kernel(all_h_list, all_t_list, user_emb, item_emb, W1, b1, W2, b2,
           uCW, uCb, uPW, uPb, uD1W, uD1b, uD2W, uD2b,
           vCW, vCb, vPW, vPb, vD1W, vD1b, vD2W, vD2b):
    h = all_h_list
    t = all_t_list
    zn = jnp.zeros((NP,), _f32)
    zr = jnp.zeros((RPT, D), _f32)

    deg2 = _deg(h, zn)
    deg = deg2[0] + deg2[1]
    dinv = jnp.where(deg > 0, lax.rsqrt(jnp.maximum(deg, 1e-30)), 0.0)

    uw = (uCW, uCb.reshape(1, -1), uPW, uPb.reshape(1, -1),
          uD1W, uD1b.reshape(1, -1), uD2W, uD2b.reshape(1, -1))
    vw = (vCW, vCb.reshape(1, -1), vPW, vPb.reshape(1, -1),
          vD1W, vD1b.reshape(1, -1), vD2W, vD2b.reshape(1, -1))

    x = jnp.concatenate([user_emb, item_emb], axis=0)
    final = x
    gnn_l, int_l, iaa_l = [], [], []
    cor = jnp.float32(0.0)

    for _ in range(2):
        int_layer, a_mat, b_mat, c_mat, cor2 = _dae_precompute(
            x, uw, vw, W1, b1.reshape(1, -1), W2, b2.reshape(1, -1))
        cor = cor + cor2[0, 0] / (N_USERS * D) + cor2[0, 1] / (N_ITEMS * D)
        c_pad = jnp.pad(c_mat[:, 0], (0, NP - N))
        alpha, dsum2 = _alpha(a_mat, b_mat, c_pad, W2[:, 0], h, t, zn)
        dsum = dsum2[0] + dsum2[1]
        dsinv = jnp.where(dsum != 0, 1.0 / jnp.where(dsum != 0, dsum, 1.0),
                          0.0)
        out2 = _spmm(x, h, t, alpha, dinv, zr)
        gnn = dinv[:N, None] * out2[0, :N]
        iaa = dsinv[:N, None] * out2[1, :N]
        gnn_l.append(gnn)
        int_l.append(int_layer)
        iaa_l.append(iaa)
        x = gnn + iaa + x
        final = final + x

    return (jnp.stack(gnn_l), jnp.stack(int_l), jnp.stack(iaa_l), final, cor)


# speedup vs baseline: 8.6156x; 1.1846x over previous
import functools

import jax
import jax.numpy as jnp
from jax import lax
from jax.experimental import pallas as pl
from jax.experimental.pallas import tpu as pltpu
from jax.experimental.pallas import tpu_sc as plsc

N_USERS = 4000
N_ITEMS = 6000
N = N_USERS + N_ITEMS
NP = 10240
D = 128
E = 320000
NC, NS = 2, 16
NW = NC * NS
CH = 80
EPW = E // NW
EPS = E // NS
RPT = NP // NS

_MESH = dict(core_axis_name="c", subcore_axis_name="s", num_cores=NC,
             num_subcores=NS)
_f32 = jnp.float32


_RB = 1000


def _dae_body(x_ref, ucw, ucb, upw, upb, ud1w, ud1b, ud2w, ud2b,
              vcw, vcb, vpw, vpb, vd1w, vd1b, vd2w, vd2b,
              w1, b1, w2, b2,
              int_ref, a_ref, b_ref, c_ref, cor_ref):
    i = pl.program_id(0)
    is_u = i < (N_USERS // _RB)
    x = x_ref[...]

    def pick(u, v):
        return jnp.where(is_u, u[...], v[...])

    cw, cb = pick(ucw, vcw), pick(ucb, vcb)
    pw, pb = pick(upw, vpw), pick(upb, vpb)
    d1w, d1b = pick(ud1w, vd1w), pick(ud1b, vd1b)
    d2w, d2b = pick(ud2w, vd2w), pick(ud2b, vd2b)

    c_ = jnp.dot(x, cw, preferred_element_type=_f32) + cb
    p_ = jnp.dot(x, pw, preferred_element_type=_f32) + pb
    r = jnp.dot(jnp.concatenate([c_, p_], axis=1), d1w,
                preferred_element_type=_f32) + d1b
    r = jnp.dot(jnp.maximum(r, 0.0), d2w, preferred_element_type=_f32) + d2b
    intb = r + x
    int_ref[...] = intb

    d2 = jnp.sum((r - x) ** 2)
    lane = lax.broadcasted_iota(jnp.int32, (1, 2), 1)
    contrib = jnp.where(lane == jnp.where(is_u, 0, 1), d2, 0.0)

    @pl.when(i == 0)
    def _():
        cor_ref[...] = jnp.zeros_like(cor_ref)

    cor_ref[...] += contrib

    nrm = jnp.sqrt(jnp.sum(intb * intb, axis=1, keepdims=True))
    y = intb / jnp.maximum(nrm, 1e-12)
    w1v = w1[...]
    a_ref[...] = jnp.dot(y, w1v[:D], preferred_element_type=_f32) + b1[...]
    b_ref[...] = jnp.dot(y, w1v[D:], preferred_element_type=_f32)
    c_ref[...] = (jnp.dot(y, w2[...], preferred_element_type=_f32)
                  + 0.5 * b2[...])


def _dae_precompute(x, uw, vw, w1, b1, w2, b2):
    full = lambda s: pl.BlockSpec(s, lambda i: tuple(0 for _ in s))
    row = lambda k: pl.BlockSpec((_RB, k), lambda i: (i, 0))
    in_specs = [row(D)]
    for w in (uw + vw):
        in_specs.append(full(w.shape))
    in_specs += [full(w1.shape), full(b1.shape), full(w2.shape),
                 full(b2.shape)]
    out_shape = (
        jax.ShapeDtypeStruct((N, D), _f32),
        jax.ShapeDtypeStruct((N, D), _f32),
        jax.ShapeDtypeStruct((N, D), _f32),
        jax.ShapeDtypeStruct((N, 1), _f32),
        jax.ShapeDtypeStruct((1, 2), _f32),
    )
    out_specs = (row(D), row(D), row(D), row(1),
                 pl.BlockSpec((1, 2), lambda i: (0, 0)))
    return pl.pallas_call(
        _dae_body, grid=(N // _RB,), in_specs=in_specs,
        out_specs=out_specs, out_shape=out_shape,
    )(x, *uw, *vw, w1, b1, w2, b2)


def _deg_body(h_hbm, zn_hbm, deg2_hbm, hv, ones_v, acc):
    cid = lax.axis_index("c")
    sid = lax.axis_index("s")
    wid = sid * NC + cid

    for g in range(CH // 16):
        ones_v[pl.ds(g * 16, 16)] = jnp.ones((16,), _f32)
    pltpu.sync_copy(zn_hbm.at[pl.ds(sid * RPT, RPT)],
                    acc.at[pl.ds(sid * RPT, RPT)])
    plsc.subcore_barrier()

    def chunk(i, _):
        base = wid * EPW + i * CH
        pltpu.sync_copy(h_hbm.at[pl.ds(base, CH)], hv)
        pltpu.sync_copy(ones_v, acc.at[hv], add=True)
        return 0

    lax.fori_loop(0, EPW // CH, chunk, 0)
    plsc.subcore_barrier()
    pltpu.sync_copy(acc.at[pl.ds(sid * RPT, RPT)],
                    deg2_hbm.at[cid, pl.ds(sid * RPT, RPT)])


def _deg(h, zn):
    return pl.kernel(
        _deg_body,
        out_type=jax.ShapeDtypeStruct((NC, NP), _f32),
        mesh=plsc.VectorSubcoreMesh(**_MESH),
        scratch_types=[
            pltpu.VMEM((CH,), jnp.int32),
            pltpu.VMEM((CH,), _f32),
            pltpu.VMEM_SHARED((NP,), _f32),
        ],
    )(h, zn)


def _alpha_body(a_hbm, b_hbm, c_hbm, w_hbm, h_hbm, t_hbm, zn_hbm,
                alpha_hbm, dsum2_hbm,
                hv0, tv0, arows0, brows0, chv0, ctv0, av0,
                hv1, tv1, arows1, brows1, chv1, ctv1, av1,
                wv, acc,
                sa0, sb0, sc0, sd0, sa1, sb1, sc1, sd1):
    cid = lax.axis_index("c")
    sid = lax.axis_index("s")
    wid = sid * NC + cid

    pltpu.sync_copy(w_hbm, wv)
    pltpu.sync_copy(zn_hbm.at[pl.ds(sid * RPT, RPT)],
                    acc.at[pl.ds(sid * RPT, RPT)])
    plsc.subcore_barrier()

    lane = lax.iota(jnp.int32, 16)
    w06 = [wv[pl.ds(j * 16, 16)] * 0.6 for j in range(D // 16)]
    w04 = [wv[pl.ds(j * 16, 16)] * 0.4 for j in range(D // 16)]
    bufs = ((hv0, tv0, arows0, brows0, chv0, ctv0, av0, sa0, sb0, sc0, sd0),
            (hv1, tv1, arows1, brows1, chv1, ctv1, av1, sa1, sb1, sc1, sd1))
    nch = EPW // CH

    def load_fire(ci, b):
        hv, tv, arows, brows, chv, ctv, av, sa, sb, sc, sd = b
        base = wid * EPW + ci * CH
        pltpu.sync_copy(h_hbm.at[pl.ds(base, CH)], hv)
        pltpu.sync_copy(t_hbm.at[pl.ds(base, CH)], tv)
        pltpu.async_copy(a_hbm.at[hv], arows, sa)
        pltpu.async_copy(b_hbm.at[tv], brows, sb)
        pltpu.async_copy(c_hbm.at[hv], chv, sc)
        pltpu.async_copy(c_hbm.at[tv], ctv, sd)

    def process(ci, b):
        hv, tv, arows, brows, chv, ctv, av, sa, sb, sc, sd = b
        base = wid * EPW + ci * CH
        pltpu.make_async_copy(a_hbm.at[hv], arows, sa).wait()
        pltpu.make_async_copy(b_hbm.at[tv], brows, sb).wait()
        pltpu.make_async_copy(c_hbm.at[hv], chv, sc).wait()
        pltpu.make_async_copy(c_hbm.at[tv], ctv, sd).wait()

        def group(g, _):
            s16 = jnp.zeros((16,), _f32)
            for e16 in range(16):
                e = g * 16 + e16
                vacc = jnp.zeros((16,), _f32)
                for j in range(D // 16):
                    z = (arows[e, pl.ds(j * 16, 16)]
                         + brows[e, pl.ds(j * 16, 16)])
                    vacc = vacc + z * w06[j] + jnp.abs(z) * w04[j]
                for sh in (8, 4, 2, 1):
                    perm = jnp.bitwise_and(lane + sh, 15)
                    vacc = vacc + vacc.at[perm].get(mode="promise_in_bounds")
                s16 = jnp.where(lane == e16, vacc, s16)
            sl = pl.ds(g * 16, 16)
            s16 = s16 + chv[sl] + ctv[sl]
            av[sl] = 1.0 / (1.0 + jnp.exp(-s16))
            return 0

        lax.fori_loop(0, CH // 16, group, 0)
        pltpu.sync_copy(av, alpha_hbm.at[pl.ds(base, CH)])
        pltpu.sync_copy(av, acc.at[hv], add=True)

    load_fire(0, bufs[0])

    def pair(k, _):
        load_fire(2 * k + 1, bufs[1])
        process(2 * k, bufs[0])
        load_fire(2 * k + 2, bufs[0])
        process(2 * k + 1, bufs[1])
        return 0

    lax.fori_loop(0, nch // 2, pair, 0)
    process(nch - 1, bufs[0])
    plsc.subcore_barrier()
    pltpu.sync_copy(acc.at[pl.ds(sid * RPT, RPT)],
                    dsum2_hbm.at[cid, pl.ds(sid * RPT, RPT)])


def _alpha(a, b, c, w, h, t, zn):
    buf = [
        pltpu.VMEM((CH,), jnp.int32),
        pltpu.VMEM((CH,), jnp.int32),
        pltpu.VMEM((CH, D), _f32),
        pltpu.VMEM((CH, D), _f32),
        pltpu.VMEM((CH,), _f32),
        pltpu.VMEM((CH,), _f32),
        pltpu.VMEM((CH,), _f32),
    ]
    return pl.kernel(
        _alpha_body,
        out_type=(jax.ShapeDtypeStruct((E,), _f32),
                  jax.ShapeDtypeStruct((NC, NP), _f32)),
        mesh=plsc.VectorSubcoreMesh(**_MESH),
        scratch_types=buf + buf + [
            pltpu.VMEM((D,), _f32),
            pltpu.VMEM_SHARED((NP,), _f32),
        ] + [pltpu.SemaphoreType.DMA] * 8,
    )(a, b, c, w, h, t, zn)


def _spmm_body(x_hbm, h_hbm, t_hbm, alpha_hbm, dinv_hbm, zr_hbm,
               out_hbm,
               hv0, tv0, av0, gt0, rows0, valv0,
               hv1, tv1, av1, gt1, rows1, valv1,
               acc, semr0, semg0, semr1, semg1):
    cid = lax.axis_index("c")
    sid = lax.axis_index("s")

    pltpu.sync_copy(zr_hbm, acc.at[pl.ds(sid * RPT, RPT)])
    plsc.subcore_barrier()
    is0 = cid == 0
    bufs = ((hv0, tv0, av0, gt0, rows0, valv0, semr0, semg0),
            (hv1, tv1, av1, gt1, rows1, valv1, semr1, semg1))
    nch = EPS // CH

    def load_fire(ci, b):
        hv, tv, av, gt, rows, valv, semr, semg = b
        base = sid * EPS + ci * CH
        pltpu.sync_copy(h_hbm.at[pl.ds(base, CH)], hv)
        pltpu.sync_copy(t_hbm.at[pl.ds(base, CH)], tv)
        pltpu.sync_copy(alpha_hbm.at[pl.ds(base, CH)], av)
        pltpu.async_copy(x_hbm.at[tv], rows, semr)
        pltpu.async_copy(dinv_hbm.at[tv], gt, semg)

    def process(b):
        hv, tv, av, gt, rows, valv, semr, semg = b
        pltpu.make_async_copy(x_hbm.at[tv], rows, semr).wait()
        pltpu.make_async_copy(dinv_hbm.at[tv], gt, semg).wait()
        for g in range(CH // 16):
            sl = pl.ds(g * 16, 16)
            valv[sl] = jnp.where(is0, gt[sl], av[sl])

        def scale(e, _):
            sc = valv[pl.ds(e, 16)][0]
            for j in range(D // 16):
                rows[e, pl.ds(j * 16, 16)] = rows[e, pl.ds(j * 16, 16)] * sc
            return 0

        lax.fori_loop(0, CH, scale, 0)
        pltpu.sync_copy(rows, acc.at[hv], add=True)

    load_fire(0, bufs[0])

    def pair(k, _):
        load_fire(2 * k + 1, bufs[1])
        process(bufs[0])

        @pl.when(k < nch // 2 - 1)
        def _():
            load_fire(2 * k + 2, bufs[0])

        process(bufs[1])
        return 0

    lax.fori_loop(0, nch // 2, pair, 0)
    plsc.subcore_barrier()
    pltpu.sync_copy(acc.at[pl.ds(sid * RPT, RPT)],
                    out_hbm.at[cid, pl.ds(sid * RPT, RPT)])


def _spmm(x, h, t, alpha, dinv, zr):
    buf = [
        pltpu.VMEM((CH,), jnp.int32),
        pltpu.VMEM((CH,), jnp.int32),
        pltpu.VMEM((CH,), _f32),
        pltpu.VMEM((CH,), _f32),
        pltpu.VMEM((CH, D), _f32),
        pltpu.VMEM((CH + 16,), _f32),
    ]
    return pl.kernel(
        _spmm_body,
        out_type=jax.ShapeDtypeStruct((NC, NP, D), _f32),
        mesh=plsc.VectorSubcoreMesh(**_MESH),
        scratch_types=buf + buf + [
            pltpu.VMEM_SHARED((NP, D), _f32),
            pltpu.SemaphoreType.DMA,
            pltpu.SemaphoreType.DMA,
            pltpu.SemaphoreType.DMA,
            pltpu.SemaphoreType.DMA,
        ],
    )(x, h, t, alpha, dinv, zr)


def kernel(all_h_list, all_t_list, user_emb, item_emb, W1, b1, W2, b2,
           uCW, uCb, uPW, uPb, uD1W, uD1b, uD2W, uD2b,
           vCW, vCb, vPW, vPb, vD1W, vD1b, vD2W, vD2b):
    h = all_h_list
    t = all_t_list
    zn = jnp.zeros((NP,), _f32)
    zr = jnp.zeros((RPT, D), _f32)

    deg2 = _deg(h, zn)
    deg = deg2[0] + deg2[1]
    dinv = jnp.where(deg > 0, lax.rsqrt(jnp.maximum(deg, 1e-30)), 0.0)

    uw = (uCW, uCb.reshape(1, -1), uPW, uPb.reshape(1, -1),
          uD1W, uD1b.reshape(1, -1), uD2W, uD2b.reshape(1, -1))
    vw = (vCW, vCb.reshape(1, -1), vPW, vPb.reshape(1, -1),
          vD1W, vD1b.reshape(1, -1), vD2W, vD2b.reshape(1, -1))

    x = jnp.concatenate([user_emb, item_emb], axis=0)
    final = x
    gnn_l, int_l, iaa_l = [], [], []
    cor = jnp.float32(0.0)

    for _ in range(2):
        int_layer, a_mat, b_mat, c_mat, cor2 = _dae_precompute(
            x, uw, vw, W1, b1.reshape(1, -1), W2, b2.reshape(1, -1))
        cor = cor + cor2[0, 0] / (N_USERS * D) + cor2[0, 1] / (N_ITEMS * D)
        c_pad = jnp.pad(c_mat[:, 0], (0, NP - N))
        alpha, dsum2 = _alpha(a_mat, b_mat, c_pad, W2[:, 0], h, t, zn)
        dsum = dsum2[0] + dsum2[1]
        dsinv = jnp.where(dsum != 0, 1.0 / jnp.where(dsum != 0, dsum, 1.0),
                          0.0)
        out2 = _spmm(x, h, t, alpha, dinv, zr)
        gnn = dinv[:N, None] * out2[0, :N]
        iaa = dsinv[:N, None] * out2[1, :N]
        gnn_l.append(gnn)
        int_l.append(int_layer)
        iaa_l.append(iaa)
        x = gnn + iaa + x
        final = final + x

    return (jnp.stack(gnn_l), jnp.stack(int_l), jnp.stack(iaa_l), final, cor)


# speedup vs baseline: 9.5131x; 1.1042x over previous
import functools

import jax
import jax.numpy as jnp
from jax import lax
from jax.experimental import pallas as pl
from jax.experimental.pallas import tpu as pltpu
from jax.experimental.pallas import tpu_sc as plsc

N_USERS = 4000
N_ITEMS = 6000
N = N_USERS + N_ITEMS
NP = 10240
D = 128
E = 320000
NC, NS = 2, 16
NW = NC * NS
CH = 80
EPW = E // NW
EPS = E // NS
RPT = NP // NS

_MESH = dict(core_axis_name="c", subcore_axis_name="s", num_cores=NC,
             num_subcores=NS)
_f32 = jnp.float32


_RB = 1000


def _dae_body(x_ref, ucw, ucb, upw, upb, ud1w, ud1b, ud2w, ud2b,
              vcw, vcb, vpw, vpb, vd1w, vd1b, vd2w, vd2b,
              w1, b1, w2, b2,
              int_ref, a_ref, b_ref, c_ref, cor_ref):
    i = pl.program_id(0)
    is_u = i < (N_USERS // _RB)
    x = x_ref[...]

    def pick(u, v):
        return jnp.where(is_u, u[...], v[...])

    cw, cb = pick(ucw, vcw), pick(ucb, vcb)
    pw, pb = pick(upw, vpw), pick(upb, vpb)
    d1w, d1b = pick(ud1w, vd1w), pick(ud1b, vd1b)
    d2w, d2b = pick(ud2w, vd2w), pick(ud2b, vd2b)

    c_ = jnp.dot(x, cw, preferred_element_type=_f32) + cb
    p_ = jnp.dot(x, pw, preferred_element_type=_f32) + pb
    r = jnp.dot(jnp.concatenate([c_, p_], axis=1), d1w,
                preferred_element_type=_f32) + d1b
    r = jnp.dot(jnp.maximum(r, 0.0), d2w, preferred_element_type=_f32) + d2b
    intb = r + x
    int_ref[...] = intb

    d2 = jnp.sum((r - x) ** 2)
    lane = lax.broadcasted_iota(jnp.int32, (1, 2), 1)
    contrib = jnp.where(lane == jnp.where(is_u, 0, 1), d2, 0.0)

    @pl.when(i == 0)
    def _():
        cor_ref[...] = jnp.zeros_like(cor_ref)

    cor_ref[...] += contrib

    nrm = jnp.sqrt(jnp.sum(intb * intb, axis=1, keepdims=True))
    y = intb / jnp.maximum(nrm, 1e-12)
    w1v = w1[...]
    a_ref[...] = jnp.dot(y, w1v[:D], preferred_element_type=_f32) + b1[...]
    b_ref[...] = jnp.dot(y, w1v[D:], preferred_element_type=_f32)
    c_ref[...] = (jnp.dot(y, w2[...], preferred_element_type=_f32)
                  + 0.5 * b2[...])


def _dae_precompute(x, uw, vw, w1, b1, w2, b2):
    full = lambda s: pl.BlockSpec(s, lambda i: tuple(0 for _ in s))
    row = lambda k: pl.BlockSpec((_RB, k), lambda i: (i, 0))
    in_specs = [row(D)]
    for w in (uw + vw):
        in_specs.append(full(w.shape))
    in_specs += [full(w1.shape), full(b1.shape), full(w2.shape),
                 full(b2.shape)]
    out_shape = (
        jax.ShapeDtypeStruct((N, D), _f32),
        jax.ShapeDtypeStruct((N, D), _f32),
        jax.ShapeDtypeStruct((N, D), _f32),
        jax.ShapeDtypeStruct((N, 1), _f32),
        jax.ShapeDtypeStruct((1, 2), _f32),
    )
    out_specs = (row(D), row(D), row(D), row(1),
                 pl.BlockSpec((1, 2), lambda i: (0, 0)))
    return pl.pallas_call(
        _dae_body, grid=(N // _RB,), in_specs=in_specs,
        out_specs=out_specs, out_shape=out_shape,
    )(x, *uw, *vw, w1, b1, w2, b2)


def _deg_body(h_hbm, zn_hbm, deg2_hbm, hv, ones_v, acc):
    cid = lax.axis_index("c")
    sid = lax.axis_index("s")
    wid = sid * NC + cid

    for g in range(CH // 16):
        ones_v[pl.ds(g * 16, 16)] = jnp.ones((16,), _f32)
    pltpu.sync_copy(zn_hbm.at[pl.ds(sid * RPT, RPT)],
                    acc.at[pl.ds(sid * RPT, RPT)])
    plsc.subcore_barrier()

    def chunk(i, _):
        base = wid * EPW + i * CH
        pltpu.sync_copy(h_hbm.at[pl.ds(base, CH)], hv)
        pltpu.sync_copy(ones_v, acc.at[hv], add=True)
        return 0

    lax.fori_loop(0, EPW // CH, chunk, 0)
    plsc.subcore_barrier()
    pltpu.sync_copy(acc.at[pl.ds(sid * RPT, RPT)],
                    deg2_hbm.at[cid, pl.ds(sid * RPT, RPT)])


def _deg(h, zn):
    return pl.kernel(
        _deg_body,
        out_type=jax.ShapeDtypeStruct((NC, NP), _f32),
        mesh=plsc.VectorSubcoreMesh(**_MESH),
        scratch_types=[
            pltpu.VMEM((CH,), jnp.int32),
            pltpu.VMEM((CH,), _f32),
            pltpu.VMEM_SHARED((NP,), _f32),
        ],
    )(h, zn)


def _alpha_body(a_hbm, b_hbm, c_hbm, w_hbm, h_hbm, t_hbm,
                alpha_hbm,
                hv0, tv0, arows0, brows0, chv0, ctv0, av0,
                hv1, tv1, arows1, brows1, chv1, ctv1, av1,
                wv,
                sa0, sb0, sc0, sd0, sa1, sb1, sc1, sd1):
    cid = lax.axis_index("c")
    sid = lax.axis_index("s")
    wid = sid * NC + cid

    pltpu.sync_copy(w_hbm, wv)
    lane = lax.iota(jnp.int32, 16)
    w06 = [wv[pl.ds(j * 16, 16)] * 0.6 for j in range(D // 16)]
    w04 = [wv[pl.ds(j * 16, 16)] * 0.4 for j in range(D // 16)]
    bufs = ((hv0, tv0, arows0, brows0, chv0, ctv0, av0, sa0, sb0, sc0, sd0),
            (hv1, tv1, arows1, brows1, chv1, ctv1, av1, sa1, sb1, sc1, sd1))
    nch = EPW // CH

    def load_fire(ci, b):
        hv, tv, arows, brows, chv, ctv, av, sa, sb, sc, sd = b
        base = wid * EPW + ci * CH
        pltpu.sync_copy(h_hbm.at[pl.ds(base, CH)], hv)
        pltpu.sync_copy(t_hbm.at[pl.ds(base, CH)], tv)
        pltpu.async_copy(a_hbm.at[hv], arows, sa)
        pltpu.async_copy(b_hbm.at[tv], brows, sb)
        pltpu.async_copy(c_hbm.at[hv], chv, sc)
        pltpu.async_copy(c_hbm.at[tv], ctv, sd)

    def process(ci, b):
        hv, tv, arows, brows, chv, ctv, av, sa, sb, sc, sd = b
        base = wid * EPW + ci * CH
        pltpu.make_async_copy(a_hbm.at[hv], arows, sa).wait()
        pltpu.make_async_copy(b_hbm.at[tv], brows, sb).wait()
        pltpu.make_async_copy(c_hbm.at[hv], chv, sc).wait()
        pltpu.make_async_copy(c_hbm.at[tv], ctv, sd).wait()

        def group(g, _):
            s16 = jnp.zeros((16,), _f32)
            for e16 in range(16):
                e = g * 16 + e16
                vacc = jnp.zeros((16,), _f32)
                for j in range(D // 16):
                    z = (arows[e, pl.ds(j * 16, 16)]
                         + brows[e, pl.ds(j * 16, 16)])
                    vacc = vacc + z * w06[j] + jnp.abs(z) * w04[j]
                for sh in (8, 4, 2, 1):
                    perm = jnp.bitwise_and(lane + sh, 15)
                    vacc = vacc + vacc.at[perm].get(mode="promise_in_bounds")
                s16 = jnp.where(lane == e16, vacc, s16)
            sl = pl.ds(g * 16, 16)
            s16 = s16 + chv[sl] + ctv[sl]
            av[sl] = 1.0 / (1.0 + jnp.exp(-s16))
            return 0

        lax.fori_loop(0, CH // 16, group, 0)
        pltpu.sync_copy(av, alpha_hbm.at[pl.ds(base, CH)])

    load_fire(0, bufs[0])

    def pair(k, _):
        load_fire(2 * k + 1, bufs[1])
        process(2 * k, bufs[0])
        load_fire(2 * k + 2, bufs[0])
        process(2 * k + 1, bufs[1])
        return 0

    lax.fori_loop(0, nch // 2, pair, 0)
    process(nch - 1, bufs[0])


def _alpha(a, b, c, w, h, t):
    buf = [
        pltpu.VMEM((CH,), jnp.int32),
        pltpu.VMEM((CH,), jnp.int32),
        pltpu.VMEM((CH, D), _f32),
        pltpu.VMEM((CH, D), _f32),
        pltpu.VMEM((CH,), _f32),
        pltpu.VMEM((CH,), _f32),
        pltpu.VMEM((CH,), _f32),
    ]
    return pl.kernel(
        _alpha_body,
        out_type=jax.ShapeDtypeStruct((E,), _f32),
        mesh=plsc.VectorSubcoreMesh(**_MESH),
        scratch_types=buf + buf + [
            pltpu.VMEM((D,), _f32),
        ] + [pltpu.SemaphoreType.DMA] * 8,
    )(a, b, c, w, h, t)


def _spmm_body(x_hbm, h_hbm, t_hbm, alpha_hbm, zr_hbm, zn_hbm,
               out_hbm, dsum_hbm,
               hv0, tv0, av0, rows0,
               hv1, tv1, av1, rows1,
               acc, dacc, semr0, semr1):
    cid = lax.axis_index("c")
    sid = lax.axis_index("s")

    pltpu.sync_copy(zr_hbm, acc.at[pl.ds(sid * RPT, RPT)])
    pltpu.sync_copy(zn_hbm.at[pl.ds(sid * RPT, RPT)],
                    dacc.at[pl.ds(sid * RPT, RPT)])
    plsc.subcore_barrier()
    toff = cid * N
    bufs = ((hv0, tv0, av0, rows0, semr0),
            (hv1, tv1, av1, rows1, semr1))
    nch = EPS // CH

    def load_fire(ci, b):
        hv, tv, av, rows, semr = b
        base = sid * EPS + ci * CH
        pltpu.sync_copy(h_hbm.at[pl.ds(base, CH)], hv)
        pltpu.sync_copy(t_hbm.at[pl.ds(base, CH)], tv)
        pltpu.sync_copy(alpha_hbm.at[pl.ds(base, CH)], av)
        for g in range(CH // 16):
            sl = pl.ds(g * 16, 16)
            tv[sl] = tv[sl] + toff
        pltpu.async_copy(x_hbm.at[tv], rows, semr)

    def process(b):
        hv, tv, av, rows, semr = b
        pltpu.make_async_copy(x_hbm.at[tv], rows, semr).wait()

        @pl.when(cid == 1)
        def _():
            def scale(g, _):
                sv = av[pl.ds(g * 16, 16)]
                for u in range(16):
                    e = g * 16 + u
                    sc = sv[u]
                    for j in range(D // 16):
                        sl = pl.ds(j * 16, 16)
                        rows[e, sl] = rows[e, sl] * sc
                return 0

            lax.fori_loop(0, CH // 16, scale, 0)

        pltpu.sync_copy(rows, acc.at[hv], add=True)
        pltpu.sync_copy(av, dacc.at[hv], add=True)

    load_fire(0, bufs[0])

    def pair(k, _):
        load_fire(2 * k + 1, bufs[1])
        process(bufs[0])

        @pl.when(k < nch // 2 - 1)
        def _():
            load_fire(2 * k + 2, bufs[0])

        process(bufs[1])
        return 0

    lax.fori_loop(0, nch // 2, pair, 0)
    plsc.subcore_barrier()
    pltpu.sync_copy(acc.at[pl.ds(sid * RPT, RPT)],
                    out_hbm.at[cid, pl.ds(sid * RPT, RPT)])
    pltpu.sync_copy(dacc.at[pl.ds(sid * RPT, RPT)],
                    dsum_hbm.at[cid, pl.ds(sid * RPT, RPT)])


def _spmm(xboth, h, t, alpha, zr, zn):
    buf = [
        pltpu.VMEM((CH,), jnp.int32),
        pltpu.VMEM((CH,), jnp.int32),
        pltpu.VMEM((CH,), _f32),
        pltpu.VMEM((CH, D), _f32),
    ]
    return pl.kernel(
        _spmm_body,
        out_type=(jax.ShapeDtypeStruct((NC, NP, D), _f32),
                  jax.ShapeDtypeStruct((NC, NP), _f32)),
        mesh=plsc.VectorSubcoreMesh(**_MESH),
        scratch_types=buf + buf + [
            pltpu.VMEM_SHARED((NP, D), _f32),
            pltpu.VMEM_SHARED((NP,), _f32),
            pltpu.SemaphoreType.DMA,
            pltpu.SemaphoreType.DMA,
        ],
    )(xboth, h, t, alpha, zr, zn)


def kernel(all_h_list, all_t_list, user_emb, item_emb, W1, b1, W2, b2,
           uCW, uCb, uPW, uPb, uD1W, uD1b, uD2W, uD2b,
           vCW, vCb, vPW, vPb, vD1W, vD1b, vD2W, vD2b):
    h = all_h_list
    t = all_t_list
    zn = jnp.zeros((NP,), _f32)
    zr = jnp.zeros((RPT, D), _f32)

    deg2 = _deg(h, zn)
    deg = deg2[0] + deg2[1]
    dinv = jnp.where(deg > 0, lax.rsqrt(jnp.maximum(deg, 1e-30)), 0.0)

    uw = (uCW, uCb.reshape(1, -1), uPW, uPb.reshape(1, -1),
          uD1W, uD1b.reshape(1, -1), uD2W, uD2b.reshape(1, -1))
    vw = (vCW, vCb.reshape(1, -1), vPW, vPb.reshape(1, -1),
          vD1W, vD1b.reshape(1, -1), vD2W, vD2b.reshape(1, -1))

    x = jnp.concatenate([user_emb, item_emb], axis=0)
    final = x
    gnn_l, int_l, iaa_l = [], [], []
    cor = jnp.float32(0.0)

    for _ in range(2):
        int_layer, a_mat, b_mat, c_mat, cor2 = _dae_precompute(
            x, uw, vw, W1, b1.reshape(1, -1), W2, b2.reshape(1, -1))
        cor = cor + cor2[0, 0] / (N_USERS * D) + cor2[0, 1] / (N_ITEMS * D)
        c_pad = jnp.pad(c_mat[:, 0], (0, NP - N))
        alpha = _alpha(a_mat, b_mat, c_pad, W2[:, 0], h, t)
        xboth = jnp.concatenate([dinv[:N, None] * x, x], axis=0)
        out2, dsum2 = _spmm(xboth, h, t, alpha, zr, zn)
        dsum = dsum2[1]
        dsinv = jnp.where(dsum != 0, 1.0 / jnp.where(dsum != 0, dsum, 1.0),
                          0.0)
        gnn = dinv[:N, None] * out2[0, :N]
        iaa = dsinv[:N, None] * out2[1, :N]
        gnn_l.append(gnn)
        int_l.append(int_layer)
        iaa_l.append(iaa)
        x = gnn + iaa + x
        final = final + x

    return (jnp.stack(gnn_l), jnp.stack(int_l), jnp.stack(iaa_l), final, cor)


# speedup vs baseline: 10.6826x; 1.1229x over previous
import functools

import jax
import jax.numpy as jnp
from jax import lax
from jax.experimental import pallas as pl
from jax.experimental.pallas import tpu as pltpu
from jax.experimental.pallas import tpu_sc as plsc

N_USERS = 4000
N_ITEMS = 6000
N = N_USERS + N_ITEMS
NP = 10240
D = 128
E = 320000
NC, NS = 2, 16
NW = NC * NS
CH = 80
EPW = E // NW
EPS = E // NS
RPT = NP // NS

_MESH = dict(core_axis_name="c", subcore_axis_name="s", num_cores=NC,
             num_subcores=NS)
_f32 = jnp.float32


_RB = 1000


def _dae_body(x_ref, ucw, ucb, upw, upb, ud1w, ud1b, ud2w, ud2b,
              vcw, vcb, vpw, vpb, vd1w, vd1b, vd2w, vd2b,
              w1, b1, w2, b2,
              int_ref, a_ref, b_ref, c_ref, cor_ref):
    i = pl.program_id(0)
    is_u = i < (N_USERS // _RB)
    x = x_ref[...]

    def pick(u, v):
        return jnp.where(is_u, u[...], v[...])

    cw, cb = pick(ucw, vcw), pick(ucb, vcb)
    pw, pb = pick(upw, vpw), pick(upb, vpb)
    d1w, d1b = pick(ud1w, vd1w), pick(ud1b, vd1b)
    d2w, d2b = pick(ud2w, vd2w), pick(ud2b, vd2b)

    c_ = jnp.dot(x, cw, preferred_element_type=_f32) + cb
    p_ = jnp.dot(x, pw, preferred_element_type=_f32) + pb
    r = jnp.dot(jnp.concatenate([c_, p_], axis=1), d1w,
                preferred_element_type=_f32) + d1b
    r = jnp.dot(jnp.maximum(r, 0.0), d2w, preferred_element_type=_f32) + d2b
    intb = r + x
    int_ref[...] = intb

    d2 = jnp.sum((r - x) ** 2)
    lane = lax.broadcasted_iota(jnp.int32, (1, 2), 1)
    contrib = jnp.where(lane == jnp.where(is_u, 0, 1), d2, 0.0)

    @pl.when(i == 0)
    def _():
        cor_ref[...] = jnp.zeros_like(cor_ref)

    cor_ref[...] += contrib

    nrm = jnp.sqrt(jnp.sum(intb * intb, axis=1, keepdims=True))
    y = intb / jnp.maximum(nrm, 1e-12)
    w1v = w1[...]
    a_ref[...] = jnp.dot(y, w1v[:D], preferred_element_type=_f32) + b1[...]
    b_ref[...] = jnp.dot(y, w1v[D:], preferred_element_type=_f32)
    c_ref[...] = (jnp.dot(y, w2[...], preferred_element_type=_f32)
                  + 0.5 * b2[...])


def _dae_precompute(x, uw, vw, w1, b1, w2, b2):
    full = lambda s: pl.BlockSpec(s, lambda i: tuple(0 for _ in s))
    row = lambda k: pl.BlockSpec((_RB, k), lambda i: (i, 0))
    in_specs = [row(D)]
    for w in (uw + vw):
        in_specs.append(full(w.shape))
    in_specs += [full(w1.shape), full(b1.shape), full(w2.shape),
                 full(b2.shape)]
    out_shape = (
        jax.ShapeDtypeStruct((N, D), _f32),
        jax.ShapeDtypeStruct((N, D), _f32),
        jax.ShapeDtypeStruct((N, D), _f32),
        jax.ShapeDtypeStruct((N, 1), _f32),
        jax.ShapeDtypeStruct((1, 2), _f32),
    )
    out_specs = (row(D), row(D), row(D), row(1),
                 pl.BlockSpec((1, 2), lambda i: (0, 0)))
    return pl.pallas_call(
        _dae_body, grid=(N // _RB,), in_specs=in_specs,
        out_specs=out_specs, out_shape=out_shape,
    )(x, *uw, *vw, w1, b1, w2, b2)


def _deg_body(h_hbm, zn_hbm, deg2_hbm, hv, ones_v, acc):
    cid = lax.axis_index("c")
    sid = lax.axis_index("s")
    wid = sid * NC + cid

    for g in range(CH // 16):
        ones_v[pl.ds(g * 16, 16)] = jnp.ones((16,), _f32)
    pltpu.sync_copy(zn_hbm.at[pl.ds(sid * RPT, RPT)],
                    acc.at[pl.ds(sid * RPT, RPT)])
    plsc.subcore_barrier()

    def chunk(i, _):
        base = wid * EPW + i * CH
        pltpu.sync_copy(h_hbm.at[pl.ds(base, CH)], hv)
        pltpu.sync_copy(ones_v, acc.at[hv], add=True)
        return 0

    lax.fori_loop(0, EPW // CH, chunk, 0)
    plsc.subcore_barrier()
    pltpu.sync_copy(acc.at[pl.ds(sid * RPT, RPT)],
                    deg2_hbm.at[cid, pl.ds(sid * RPT, RPT)])


def _deg(h, zn):
    return pl.kernel(
        _deg_body,
        out_type=jax.ShapeDtypeStruct((NC, NP), _f32),
        mesh=plsc.VectorSubcoreMesh(**_MESH),
        scratch_types=[
            pltpu.VMEM((CH,), jnp.int32),
            pltpu.VMEM((CH,), _f32),
            pltpu.VMEM_SHARED((NP,), _f32),
        ],
    )(h, zn)


def _alpha_body(a_hbm, b_hbm, c_hbm, w_hbm, ht_hbm,
                alpha_hbm,
                htv0, hv0, tv0, arows0, brows0, chv0, ctv0, av0,
                htv1, hv1, tv1, arows1, brows1, chv1, ctv1, av1,
                wv,
                sa0, sb0, sc0, sd0, sa1, sb1, sc1, sd1):
    cid = lax.axis_index("c")
    sid = lax.axis_index("s")
    wid = sid * NC + cid

    pltpu.sync_copy(w_hbm, wv)
    lane = lax.iota(jnp.int32, 16)
    w06 = [wv[pl.ds(j * 16, 16)] * 0.6 for j in range(D // 16)]
    w04 = [wv[pl.ds(j * 16, 16)] * 0.4 for j in range(D // 16)]
    bufs = ((htv0, hv0, tv0, arows0, brows0, chv0, ctv0, av0,
             sa0, sb0, sc0, sd0),
            (htv1, hv1, tv1, arows1, brows1, chv1, ctv1, av1,
             sa1, sb1, sc1, sd1))
    nch = EPW // CH

    def load_fire(ci, b):
        htv, hv, tv, arows, brows, chv, ctv, av, sa, sb, sc, sd = b
        cig = wid * (EPW // CH) + ci
        pltpu.sync_copy(ht_hbm.at[cig], htv)
        for g in range(CH // 16):
            sl = pl.ds(g * 16, 16)
            hv[sl] = htv[pl.ds(g * 16, 16)]
            tv[sl] = htv[pl.ds(CH + g * 16, 16)]
        pltpu.async_copy(a_hbm.at[hv], arows, sa)
        pltpu.async_copy(b_hbm.at[tv], brows, sb)
        pltpu.async_copy(c_hbm.at[hv], chv, sc)
        pltpu.async_copy(c_hbm.at[tv], ctv, sd)

    def process(ci, b):
        htv, hv, tv, arows, brows, chv, ctv, av, sa, sb, sc, sd = b
        base = wid * EPW + ci * CH
        pltpu.make_async_copy(a_hbm.at[hv], arows, sa).wait()
        pltpu.make_async_copy(b_hbm.at[tv], brows, sb).wait()
        pltpu.make_async_copy(c_hbm.at[hv], chv, sc).wait()
        pltpu.make_async_copy(c_hbm.at[tv], ctv, sd).wait()

        def group(g, _):
            s16 = jnp.zeros((16,), _f32)
            for e16 in range(16):
                e = g * 16 + e16
                vacc = jnp.zeros((16,), _f32)
                for j in range(D // 16):
                    z = (arows[e, pl.ds(j * 16, 16)]
                         + brows[e, pl.ds(j * 16, 16)])
                    vacc = vacc + z * w06[j] + jnp.abs(z) * w04[j]
                for sh in (8, 4, 2, 1):
                    perm = jnp.bitwise_and(lane + sh, 15)
                    vacc = vacc + vacc.at[perm].get(mode="promise_in_bounds")
                s16 = jnp.where(lane == e16, vacc, s16)
            sl = pl.ds(g * 16, 16)
            s16 = s16 + chv[sl] + ctv[sl]
            av[sl] = 1.0 / (1.0 + jnp.exp(-s16))
            return 0

        lax.fori_loop(0, CH // 16, group, 0)
        pltpu.sync_copy(av, alpha_hbm.at[pl.ds(base, CH)])

    load_fire(0, bufs[0])

    def pair(k, _):
        load_fire(2 * k + 1, bufs[1])
        process(2 * k, bufs[0])
        load_fire(2 * k + 2, bufs[0])
        process(2 * k + 1, bufs[1])
        return 0

    lax.fori_loop(0, nch // 2, pair, 0)
    process(nch - 1, bufs[0])


def _alpha(a, b, c, w, ht):
    buf = [
        pltpu.VMEM((256,), jnp.int32),
        pltpu.VMEM((CH,), jnp.int32),
        pltpu.VMEM((CH,), jnp.int32),
        pltpu.VMEM((CH, D), _f32),
        pltpu.VMEM((CH, D), _f32),
        pltpu.VMEM((CH,), _f32),
        pltpu.VMEM((CH,), _f32),
        pltpu.VMEM((CH,), _f32),
    ]
    return pl.kernel(
        _alpha_body,
        out_type=jax.ShapeDtypeStruct((E,), _f32),
        mesh=plsc.VectorSubcoreMesh(**_MESH),
        scratch_types=buf + buf + [
            pltpu.VMEM((D,), _f32),
        ] + [pltpu.SemaphoreType.DMA] * 8,
    )(a, b, c, w, ht)


def _spmm_body(x_hbm, hta_hbm, al_hbm, zr_hbm, zn_hbm,
               out_hbm, dsum_hbm,
               htv0, av128_0, hv0, tv0, av0, rows0,
               htv1, av128_1, hv1, tv1, av1, rows1,
               acc, dacc, semr0, sems0, semd0, semr1, sems1, semd1):
    cid = lax.axis_index("c")
    sid = lax.axis_index("s")

    pltpu.sync_copy(zr_hbm, acc.at[pl.ds(sid * RPT, RPT)])
    pltpu.sync_copy(zn_hbm.at[pl.ds(sid * RPT, RPT)],
                    dacc.at[pl.ds(sid * RPT, RPT)])
    plsc.subcore_barrier()
    toff = cid * N
    bufs = ((htv0, av128_0, hv0, tv0, av0, rows0, semr0, sems0, semd0),
            (htv1, av128_1, hv1, tv1, av1, rows1, semr1, sems1, semd1))
    nch = EPS // CH

    def wait_scatters(b):
        htv, av128, hv, tv, av, rows, semr, sems, semd = b
        pltpu.make_async_copy(rows, acc.at[hv], sems).wait()
        pltpu.make_async_copy(av, dacc.at[hv], semd).wait()

    def load_fire(ci, b):
        htv, av128, hv, tv, av, rows, semr, sems, semd = b
        cig = sid * (EPS // CH) + ci
        pltpu.sync_copy(hta_hbm.at[cig], htv)
        pltpu.sync_copy(al_hbm.at[cig], av128)
        for g in range(CH // 16):
            sl = pl.ds(g * 16, 16)
            hv[sl] = htv[pl.ds(g * 16, 16)]
            tv[sl] = htv[pl.ds(CH + g * 16, 16)] + toff
            av[sl] = av128[sl]
        pltpu.async_copy(x_hbm.at[tv], rows, semr)

    def process(b):
        htv, av128, hv, tv, av, rows, semr, sems, semd = b
        pltpu.make_async_copy(x_hbm.at[tv], rows, semr).wait()

        @pl.when(cid == 1)
        def _():
            def scale(g, _):
                sv = av[pl.ds(g * 16, 16)]
                for u in range(16):
                    e = g * 16 + u
                    sc = sv[u]
                    for j in range(D // 16):
                        sl = pl.ds(j * 16, 16)
                        rows[e, sl] = rows[e, sl] * sc
                return 0

            lax.fori_loop(0, CH // 16, scale, 0)

        pltpu.async_copy(rows, acc.at[hv], sems, add=True)
        pltpu.async_copy(av, dacc.at[hv], semd, add=True)

    load_fire(0, bufs[0])

    def pair(k, _):
        @pl.when(k > 0)
        def _():
            wait_scatters(bufs[1])

        load_fire(2 * k + 1, bufs[1])
        process(bufs[0])

        @pl.when(k < nch // 2 - 1)
        def _():
            wait_scatters(bufs[0])
            load_fire(2 * k + 2, bufs[0])

        process(bufs[1])
        return 0

    lax.fori_loop(0, nch // 2, pair, 0)
    wait_scatters(bufs[0])
    wait_scatters(bufs[1])
    plsc.subcore_barrier()
    pltpu.sync_copy(acc.at[pl.ds(sid * RPT, RPT)],
                    out_hbm.at[cid, pl.ds(sid * RPT, RPT)])
    pltpu.sync_copy(dacc.at[pl.ds(sid * RPT, RPT)],
                    dsum_hbm.at[cid, pl.ds(sid * RPT, RPT)])


def _spmm(xboth, hta, al4, zr, zn):
    buf = [
        pltpu.VMEM((256,), jnp.int32),
        pltpu.VMEM((128,), _f32),
        pltpu.VMEM((CH,), jnp.int32),
        pltpu.VMEM((CH,), jnp.int32),
        pltpu.VMEM((CH,), _f32),
        pltpu.VMEM((CH, D), _f32),
    ]
    sems = [pltpu.SemaphoreType.DMA] * 3
    return pl.kernel(
        _spmm_body,
        out_type=(jax.ShapeDtypeStruct((NC, NP, D), _f32),
                  jax.ShapeDtypeStruct((NC, NP), _f32)),
        mesh=plsc.VectorSubcoreMesh(**_MESH),
        scratch_types=buf + buf + [
            pltpu.VMEM_SHARED((NP, D), _f32),
            pltpu.VMEM_SHARED((NP,), _f32),
        ] + sems + sems,
    )(xboth, hta, al4, zr, zn)


def kernel(all_h_list, all_t_list, user_emb, item_emb, W1, b1, W2, b2,
           uCW, uCb, uPW, uPb, uD1W, uD1b, uD2W, uD2b,
           vCW, vCb, vPW, vPb, vD1W, vD1b, vD2W, vD2b):
    h = all_h_list
    t = all_t_list
    nchk = E // CH
    hc = h.reshape(nchk, CH)
    tc = t.reshape(nchk, CH)
    pad96 = jnp.zeros((nchk, 256 - 2 * CH), jnp.int32)
    ht4 = jnp.concatenate([hc, tc, pad96], axis=1)
    zn = jnp.zeros((NP,), _f32)
    zr = jnp.zeros((RPT, D), _f32)

    deg2 = _deg(h, zn)
    deg = deg2[0] + deg2[1]
    dinv = jnp.where(deg > 0, lax.rsqrt(jnp.maximum(deg, 1e-30)), 0.0)

    uw = (uCW, uCb.reshape(1, -1), uPW, uPb.reshape(1, -1),
          uD1W, uD1b.reshape(1, -1), uD2W, uD2b.reshape(1, -1))
    vw = (vCW, vCb.reshape(1, -1), vPW, vPb.reshape(1, -1),
          vD1W, vD1b.reshape(1, -1), vD2W, vD2b.reshape(1, -1))

    x = jnp.concatenate([user_emb, item_emb], axis=0)
    final = x
    gnn_l, int_l, iaa_l = [], [], []
    cor = jnp.float32(0.0)

    for _ in range(2):
        int_layer, a_mat, b_mat, c_mat, cor2 = _dae_precompute(
            x, uw, vw, W1, b1.reshape(1, -1), W2, b2.reshape(1, -1))
        cor = cor + cor2[0, 0] / (N_USERS * D) + cor2[0, 1] / (N_ITEMS * D)
        c_pad = jnp.pad(c_mat[:, 0], (0, NP - N))
        alpha = _alpha(a_mat, b_mat, c_pad, W2[:, 0], ht4)
        al4 = jnp.pad(alpha.reshape(nchk, CH), ((0, 0), (0, 128 - CH)))
        xboth = jnp.concatenate([dinv[:N, None] * x, x], axis=0)
        out2, dsum2 = _spmm(xboth, ht4, al4, zr, zn)
        dsum = dsum2[1]
        dsinv = jnp.where(dsum != 0, 1.0 / jnp.where(dsum != 0, dsum, 1.0),
                          0.0)
        gnn = dinv[:N, None] * out2[0, :N]
        iaa = dsinv[:N, None] * out2[1, :N]
        gnn_l.append(gnn)
        int_l.append(int_layer)
        iaa_l.append(iaa)
        x = gnn + iaa + x
        final = final + x

    return (jnp.stack(gnn_l), jnp.stack(int_l), jnp.stack(iaa_l), final, cor)
